# baseline (speedup 1.0000x reference)
import functools
import math

import jax
import jax.numpy as jnp
from jax import lax
from jax.experimental import pallas as pl
from jax.experimental.pallas import tpu as pltpu
from jax.experimental.pallas import tpu_sc as plsc

D_MODEL = 64
VOCAB = 1000000
BATCH = 4096
SEQ = 200
SCALE = math.sqrt(D_MODEL)

_INFO = plsc.get_sparse_core_info()
_NC, _NS, _L = _INFO.num_cores, _INFO.num_subcores, _INFO.num_lanes
_NW = _NC * _NS

_B = BATCH * SEQ
_B_PER_W = _B // _NW
_CHUNK = 128
_N_CHUNKS = _B_PER_W // _CHUNK
_D2 = 2 * D_MODEL
_V2 = VOCAB // 2

_N_FULL = VOCAB // 128
_KP = 4
_N_SUP = _N_FULL // _KP
_SUP_BASE = _N_SUP // _NW
_SUP_REM = _N_SUP % _NW
_TAIL_V0 = _N_FULL * 128
_TAIL_J0 = _TAIL_V0 // 2


def _tr_kernel(tT_hbm, t2_hbm, panel, pout, ptail, sem_i):
    wid = lax.axis_index("s") * _NC + lax.axis_index("c")
    iota = lax.iota(jnp.int32, _L)
    cnt = jnp.where(wid < _SUP_REM, _SUP_BASE + 1, _SUP_BASE)
    start = wid * _SUP_BASE + jnp.minimum(wid, _SUP_REM)

    def fire_in(ps, b):
        v0 = pl.multiple_of(ps * (_KP * 128), 128)
        pltpu.async_copy(
            tT_hbm.at[:, pl.ds(v0, _KP * 128)], panel.at[b], sem_i
        )

    def wait_in(b):
        pltpu.make_async_copy(
            tT_hbm.at[:, pl.ds(0, _KP * 128)], panel.at[b], sem_i
        ).wait()

    def transpose_super(b):
        def body(i, _):
            kp = lax.shift_right_logical(i, jnp.int32(2))
            g = lax.bitwise_and(i, jnp.int32(3))
            jj = g * jnp.int32(_L) + iota
            rowo = kp * jnp.int32(64) + jj
            cole = kp * jnp.int32(128) + 2 * jj
            colo = cole + 1
            pb = panel.at[b]
            for c in range(_D2):
                d = c & 63
                srccol = cole if c < D_MODEL else colo
                v = plsc.load_gather(
                    pb, [jnp.full((_L,), d, jnp.int32), srccol]
                )
                plsc.store_scatter(
                    pout, [rowo, jnp.full((_L,), c, jnp.int32)], v
                )
            return None

        lax.fori_loop(0, _KP * 4, body, None)

    def super_step(ps_local, _):
        ps = start + ps_local
        b = lax.bitwise_and(ps_local, jnp.int32(1))

        @pl.when(ps_local + 1 < cnt)
        def _():
            fire_in(ps + 1, 1 - b)

        @pl.when(b == 0)
        def _():
            transpose_super(0)

        @pl.when(b == 1)
        def _():
            transpose_super(1)

        j0 = pl.multiple_of(ps * (_KP * 64), 8)
        pltpu.sync_copy(pout, t2_hbm.at[pl.ds(j0, _KP * 64)])

        @pl.when(ps_local + 1 < cnt)
        def _():
            wait_in(1 - b)

        return None

    fire_in(start, 0)
    wait_in(0)
    lax.fori_loop(0, cnt, super_step, None)

    @pl.when(wid == _NW - 1)
    def _():
        pltpu.sync_copy(tT_hbm.at[:, pl.ds(_TAIL_V0, 64)], ptail)

        def tbody(g, _):
            jj = g * jnp.int32(_L) + iota
            cole = 2 * jj
            for c in range(_D2):
                d = c & 63
                srccol = cole if c < D_MODEL else cole + 1
                v = plsc.load_gather(
                    ptail, [jnp.full((_L,), d, jnp.int32), srccol]
                )
                plsc.store_scatter(
                    pout, [jj, jnp.full((_L,), c, jnp.int32)], v
                )
            return None

        lax.fori_loop(0, 2, tbody, None)
        pltpu.sync_copy(
            pout.at[pl.ds(0, 32)], t2_hbm.at[pl.ds(_TAIL_J0, 32)]
        )


def _emb_kernel(x_hbm, table_hbm, out_hbm, idxb, jdx, hoff, rows, comp,
                sem_i0, sem_i1, sem_g0, sem_g1, sem_o0, sem_o1):
    wid = lax.axis_index("s") * _NC + lax.axis_index("c")
    base = wid * _B_PER_W
    iota = lax.iota(jnp.int32, _L)
    sem_i = (sem_i0, sem_i1)
    sem_g = (sem_g0, sem_g1)
    sem_o = (sem_o0, sem_o1)

    def fire_idx(g, b):
        off = pl.multiple_of(base + g * _CHUNK, 8)
        pltpu.async_copy(x_hbm.at[pl.ds(off, _CHUNK)], idxb.at[b], sem_i[b])

    def wait_idx(b):
        pltpu.make_async_copy(
            x_hbm.at[pl.ds(0, _CHUNK)], idxb.at[b], sem_i[b]
        ).wait()

    def prep_idx(b):
        def body(v, _):
            sl = pl.ds(v * _L, _L)
            raw = idxb[b, sl]
            jdx[b, sl] = lax.shift_right_logical(raw, jnp.int32(1))
            hoff[b, sl] = lax.bitwise_and(raw, jnp.int32(1)) * jnp.int32(D_MODEL)
            return None

        lax.fori_loop(0, _CHUNK // _L, body, None)

    def fire_gather(b):
        pltpu.async_copy(
            table_hbm.at[jdx.at[b]], rows.at[b], sem_g[b]
        )

    def wait_gather(b):
        pltpu.make_async_copy(
            table_hbm.at[pl.ds(0, _CHUNK)], rows.at[b], sem_g[b]
        ).wait()

    def fire_writeout(g, b):
        off = pl.multiple_of(base + g * _CHUNK, 8)
        pltpu.async_copy(comp.at[b], out_hbm.at[pl.ds(off, _CHUNK)], sem_o[b])

    def wait_writeout(b):
        pltpu.make_async_copy(
            comp.at[b], out_hbm.at[pl.ds(0, _CHUNK)], sem_o[b]
        ).wait()

    def select_scale_chunk(b):
        rr = rows.at[b]
        cc = comp.at[b]

        def body(r16, _):
            r0 = r16 * _L
            rids = r0 + iota
            half16 = hoff[b, pl.ds(r0, _L)]
            for c in range(D_MODEL):
                v = plsc.load_gather(rr, [rids, half16 + c])
                plsc.store_scatter(
                    cc, [rids, jnp.full((_L,), c, jnp.int32)], v * SCALE
                )
            return None

        lax.fori_loop(0, _CHUNK // _L, body, None)

    fire_idx(0, 0)
    fire_idx(1, 1)
    wait_idx(0)
    prep_idx(0)
    fire_gather(0)

    def pair_body(gg, _):
        for b in (0, 1):
            g = gg * 2 + b

            @pl.when(g >= 1)
            def _():
                wait_writeout(1 - b)

            @pl.when(g + 1 < _N_CHUNKS)
            def _():
                wait_idx(1 - b)
                prep_idx(1 - b)
                fire_gather(1 - b)

            @pl.when(g + 2 < _N_CHUNKS)
            def _():
                fire_idx(g + 2, b)

            wait_gather(b)
            select_scale_chunk(b)
            fire_writeout(g, b)
        return None

    lax.fori_loop(0, _N_CHUNKS // 2, pair_body, None)
    wait_writeout(1)


@jax.jit
def _embed(x1d, tT):
    mesh = plsc.VectorSubcoreMesh(core_axis_name="c", subcore_axis_name="s")
    t2 = functools.partial(
        pl.kernel,
        mesh=mesh,
        out_type=jax.ShapeDtypeStruct((_V2, _D2), jnp.float32),
        scratch_types=[
            pltpu.VMEM((2, D_MODEL, _KP * 128), jnp.float32),
            pltpu.VMEM((_KP * 64, _D2), jnp.float32),
            pltpu.VMEM((D_MODEL, 64), jnp.float32),
            pltpu.SemaphoreType.DMA,
        ],
        compiler_params=pltpu.CompilerParams(use_tc_tiling_on_sc=True, needs_layout_passes=False),
    )(_tr_kernel)(tT)
    out = functools.partial(
        pl.kernel,
        mesh=mesh,
        out_type=jax.ShapeDtypeStruct((_B, D_MODEL), jnp.float32),
        scratch_types=[
            pltpu.VMEM((2, _CHUNK), jnp.int32),
            pltpu.VMEM((2, _CHUNK), jnp.int32),
            pltpu.VMEM((2, _CHUNK), jnp.int32),
            pltpu.VMEM((2, _CHUNK, _D2), jnp.float32),
            pltpu.VMEM((2, _CHUNK, D_MODEL), jnp.float32),
            pltpu.SemaphoreType.DMA,
            pltpu.SemaphoreType.DMA,
            pltpu.SemaphoreType.DMA,
            pltpu.SemaphoreType.DMA,
            pltpu.SemaphoreType.DMA,
            pltpu.SemaphoreType.DMA,
        ],
        compiler_params=pltpu.CompilerParams(use_tc_tiling_on_sc=True, needs_layout_passes=False),
    )(_emb_kernel)(x1d, t2)
    return out


def kernel(x, table):
    x1d = x.reshape(_B).astype(jnp.int32)
    out = _embed(x1d, table.T)
    return out.reshape(BATCH, SEQ, D_MODEL)


# speedup vs baseline: 1.2401x; 1.2401x over previous
import functools
import math

import jax
import jax.numpy as jnp
from jax import lax
from jax.experimental import pallas as pl
from jax.experimental.pallas import tpu as pltpu
from jax.experimental.pallas import tpu_sc as plsc

D_MODEL = 64
VOCAB = 1000000
BATCH = 4096
SEQ = 200
SCALE = math.sqrt(D_MODEL)

_INFO = plsc.get_sparse_core_info()
_NC, _NS, _L = _INFO.num_cores, _INFO.num_subcores, _INFO.num_lanes
_NW = _NC * _NS

_B = BATCH * SEQ
_B_PER_W = _B // _NW
_CHUNK = 128
_N_CHUNKS = _B_PER_W // _CHUNK
_D2 = 2 * D_MODEL
_V2 = VOCAB // 2

_N_FULL = VOCAB // 128
_KP = 4
_N_SUP = _N_FULL // _KP
_SUP_BASE = _N_SUP // _NW
_SUP_REM = _N_SUP % _NW
_TAIL_V0 = _N_FULL * 128
_TAIL_J0 = _TAIL_V0 // 2


def _tr_kernel(tT_hbm, t2_hbm, panel, pout, ptail, sem_i):
    wid = lax.axis_index("s") * _NC + lax.axis_index("c")
    iota = lax.iota(jnp.int32, _L)
    cnt = jnp.where(wid < _SUP_REM, _SUP_BASE + 1, _SUP_BASE)
    start = wid * _SUP_BASE + jnp.minimum(wid, _SUP_REM)

    def fire_in(ps, b):
        v0 = pl.multiple_of(ps * (_KP * 128), 128)
        pltpu.async_copy(
            tT_hbm.at[:, pl.ds(v0, _KP * 128)], panel.at[b], sem_i
        )

    def wait_in(b):
        pltpu.make_async_copy(
            tT_hbm.at[:, pl.ds(0, _KP * 128)], panel.at[b], sem_i
        ).wait()

    def transpose_super(b):
        @plsc.parallel_loop(0, _KP * 4, unroll=2)
        def body(i):
            kp = lax.shift_right_logical(i, jnp.int32(2))
            g = lax.bitwise_and(i, jnp.int32(3))
            jj = g * jnp.int32(_L) + iota
            rowo = kp * jnp.int32(64) + jj
            cole = kp * jnp.int32(128) + 2 * jj
            colo = cole + 1
            pb = panel.at[b]
            for c in range(_D2):
                d = c & 63
                srccol = cole if c < D_MODEL else colo
                v = plsc.load_gather(
                    pb, [jnp.full((_L,), d, jnp.int32), srccol]
                )
                plsc.store_scatter(
                    pout, [rowo, jnp.full((_L,), c, jnp.int32)], v * SCALE
                )

    def super_step(ps_local, _):
        ps = start + ps_local
        b = lax.bitwise_and(ps_local, jnp.int32(1))

        @pl.when(ps_local + 1 < cnt)
        def _():
            fire_in(ps + 1, 1 - b)

        @pl.when(b == 0)
        def _():
            transpose_super(0)

        @pl.when(b == 1)
        def _():
            transpose_super(1)

        j0 = pl.multiple_of(ps * (_KP * 64), 8)
        pltpu.sync_copy(pout, t2_hbm.at[pl.ds(j0, _KP * 64)])

        @pl.when(ps_local + 1 < cnt)
        def _():
            wait_in(1 - b)

        return None

    fire_in(start, 0)
    wait_in(0)
    lax.fori_loop(0, cnt, super_step, None)

    @pl.when(wid == _NW - 1)
    def _():
        pltpu.sync_copy(tT_hbm.at[:, pl.ds(_TAIL_V0, 64)], ptail)

        @plsc.parallel_loop(0, 2)
        def tbody(g):
            jj = g * jnp.int32(_L) + iota
            cole = 2 * jj
            for c in range(_D2):
                d = c & 63
                srccol = cole if c < D_MODEL else cole + 1
                v = plsc.load_gather(
                    ptail, [jnp.full((_L,), d, jnp.int32), srccol]
                )
                plsc.store_scatter(
                    pout, [jj, jnp.full((_L,), c, jnp.int32)], v * SCALE
                )
        pltpu.sync_copy(
            pout.at[pl.ds(0, 32)], t2_hbm.at[pl.ds(_TAIL_J0, 32)]
        )


def _emb_kernel(x_hbm, table_hbm, out_hbm, idxb, jdx, hoff, rows, comp,
                sem_i0, sem_i1, sem_g0, sem_g1, sem_o0, sem_o1):
    wid = lax.axis_index("s") * _NC + lax.axis_index("c")
    base = wid * _B_PER_W
    iota = lax.iota(jnp.int32, _L)
    sem_i = (sem_i0, sem_i1)
    sem_g = (sem_g0, sem_g1)
    sem_o = (sem_o0, sem_o1)

    def fire_idx(g, b):
        off = pl.multiple_of(base + g * _CHUNK, 8)
        pltpu.async_copy(x_hbm.at[pl.ds(off, _CHUNK)], idxb.at[b], sem_i[b])

    def wait_idx(b):
        pltpu.make_async_copy(
            x_hbm.at[pl.ds(0, _CHUNK)], idxb.at[b], sem_i[b]
        ).wait()

    def prep_idx(b):
        def body(v, _):
            sl = pl.ds(v * _L, _L)
            raw = idxb[b, sl]
            jdx[b, sl] = lax.shift_right_logical(raw, jnp.int32(1))
            hoff[b, sl] = lax.bitwise_and(raw, jnp.int32(1)) * jnp.int32(D_MODEL)
            return None

        lax.fori_loop(0, _CHUNK // _L, body, None)

    def fire_gather(b):
        pltpu.async_copy(
            table_hbm.at[jdx.at[b]], rows.at[b], sem_g[b]
        )

    def wait_gather(b):
        pltpu.make_async_copy(
            table_hbm.at[pl.ds(0, _CHUNK)], rows.at[b], sem_g[b]
        ).wait()

    def fire_writeout(g, b):
        off = pl.multiple_of(base + g * _CHUNK, 8)
        pltpu.async_copy(comp.at[b], out_hbm.at[pl.ds(off, _CHUNK)], sem_o[b])

    def wait_writeout(b):
        pltpu.make_async_copy(
            comp.at[b], out_hbm.at[pl.ds(0, _CHUNK)], sem_o[b]
        ).wait()

    def select_scale_chunk(b):
        rr = rows.at[b]
        cc = comp.at[b]

        @plsc.parallel_loop(0, _CHUNK // _L, unroll=2)
        def body(r16):
            r0 = r16 * _L
            rids = r0 + iota
            half16 = hoff[b, pl.ds(r0, _L)]
            for c in range(D_MODEL):
                v = plsc.load_gather(rr, [rids, half16 + c])
                plsc.store_scatter(
                    cc, [rids, jnp.full((_L,), c, jnp.int32)], v
                )

    fire_idx(0, 0)
    fire_idx(1, 1)
    wait_idx(0)
    prep_idx(0)
    fire_gather(0)

    def pair_body(gg, _):
        for b in (0, 1):
            g = gg * 2 + b

            @pl.when(g >= 1)
            def _():
                wait_writeout(1 - b)

            @pl.when(g + 1 < _N_CHUNKS)
            def _():
                wait_idx(1 - b)
                prep_idx(1 - b)
                fire_gather(1 - b)

            @pl.when(g + 2 < _N_CHUNKS)
            def _():
                fire_idx(g + 2, b)

            wait_gather(b)
            select_scale_chunk(b)
            fire_writeout(g, b)
        return None

    lax.fori_loop(0, _N_CHUNKS // 2, pair_body, None)
    wait_writeout(1)


@jax.jit
def _embed(x1d, tT):
    mesh = plsc.VectorSubcoreMesh(core_axis_name="c", subcore_axis_name="s")
    t2 = functools.partial(
        pl.kernel,
        mesh=mesh,
        out_type=jax.ShapeDtypeStruct((_V2, _D2), jnp.float32),
        scratch_types=[
            pltpu.VMEM((2, D_MODEL, _KP * 128), jnp.float32),
            pltpu.VMEM((_KP * 64, _D2), jnp.float32),
            pltpu.VMEM((D_MODEL, 64), jnp.float32),
            pltpu.SemaphoreType.DMA,
        ],
        compiler_params=pltpu.CompilerParams(use_tc_tiling_on_sc=True, needs_layout_passes=False),
    )(_tr_kernel)(tT)
    out = functools.partial(
        pl.kernel,
        mesh=mesh,
        out_type=jax.ShapeDtypeStruct((_B, D_MODEL), jnp.float32),
        scratch_types=[
            pltpu.VMEM((2, _CHUNK), jnp.int32),
            pltpu.VMEM((2, _CHUNK), jnp.int32),
            pltpu.VMEM((2, _CHUNK), jnp.int32),
            pltpu.VMEM((2, _CHUNK, _D2), jnp.float32),
            pltpu.VMEM((2, _CHUNK, D_MODEL), jnp.float32),
            pltpu.SemaphoreType.DMA,
            pltpu.SemaphoreType.DMA,
            pltpu.SemaphoreType.DMA,
            pltpu.SemaphoreType.DMA,
            pltpu.SemaphoreType.DMA,
            pltpu.SemaphoreType.DMA,
        ],
        compiler_params=pltpu.CompilerParams(use_tc_tiling_on_sc=True, needs_layout_passes=False),
    )(_emb_kernel)(x1d, t2)
    return out


def kernel(x, table):
    x1d = x.reshape(_B).astype(jnp.int32)
    out = _embed(x1d, table.T)
    return out.reshape(BATCH, SEQ, D_MODEL)


# speedup vs baseline: 2.9241x; 2.3579x over previous
import functools
import math

import jax
import jax.numpy as jnp
from jax import lax
from jax.experimental import pallas as pl
from jax.experimental.pallas import tpu as pltpu
from jax.experimental.pallas import tpu_sc as plsc

D_MODEL = 64
VOCAB = 1000000
BATCH = 4096
SEQ = 200
SCALE = math.sqrt(D_MODEL)

_INFO = plsc.get_sparse_core_info()
_NC, _NS, _L = _INFO.num_cores, _INFO.num_subcores, _INFO.num_lanes
_NW = _NC * _NS

_B = BATCH * SEQ
_B_PER_W = _B // _NW
_BATCH_PER_W = BATCH // _NW
_ROWS_PER_CHUNK = 2
_CHUNK = _ROWS_PER_CHUNK * SEQ
_N_CHUNKS = _B_PER_W // _CHUNK
_IDX_SUB = 80
_N_SUB = _CHUNK // _IDX_SUB
_ROWS_UNROLL = 4


def _emb_kernel(x_hbm, table_hbm, out_hbm, idx_all, rows, sem_g0, sem_g1,
                sem_o0, sem_o1):
    wid = lax.axis_index("s") * _NC + lax.axis_index("c")
    base = wid * _B_PER_W
    brow0 = wid * _BATCH_PER_W
    sem_g = (sem_g0, sem_g1)
    sem_o = (sem_o0, sem_o1)

    pltpu.sync_copy(x_hbm.at[pl.ds(pl.multiple_of(base, 8), _B_PER_W)], idx_all)

    def fire_gathers(g, b):
        for k in range(_N_SUB):
            off = pl.multiple_of(g * _CHUNK + k * _IDX_SUB, 8)
            pltpu.async_copy(
                table_hbm.at[idx_all.at[pl.ds(off, _IDX_SUB)]],
                rows.at[b, pl.ds(k * _IDX_SUB, _IDX_SUB)],
                sem_g[b],
            )

    def wait_gathers(b):
        pltpu.make_async_copy(
            table_hbm.at[pl.ds(0, _CHUNK)], rows.at[b], sem_g[b]
        ).wait()

    def fire_writeout(g, b):
        br = brow0 + g * _ROWS_PER_CHUNK
        for j in range(_ROWS_PER_CHUNK):
            pltpu.async_copy(
                rows.at[b, pl.ds(j * SEQ, SEQ)], out_hbm.at[br + j], sem_o[b]
            )

    def wait_writeout(b):
        for j in range(_ROWS_PER_CHUNK):
            pltpu.make_async_copy(
                rows.at[b, pl.ds(j * SEQ, SEQ)], out_hbm.at[0], sem_o[b]
            ).wait()

    def scale_chunk(b):
        rr = rows.at[b]

        def scale_body(r4, _):
            r0 = r4 * _ROWS_UNROLL
            for dr in range(_ROWS_UNROLL):
                for c4 in range(D_MODEL // _L):
                    sl = pl.ds(c4 * _L, _L)
                    rr[r0 + dr, sl] = rr[r0 + dr, sl] * SCALE
            return None

        lax.fori_loop(0, _CHUNK // _ROWS_UNROLL, scale_body, None)

    fire_gathers(0, 0)

    def pair_body(gg, _):
        for b in (0, 1):
            g = gg * 2 + b

            @pl.when(g >= 1)
            def _():
                wait_writeout(1 - b)

            @pl.when(g + 1 < _N_CHUNKS)
            def _():
                fire_gathers(g + 1, 1 - b)

            wait_gathers(b)
            scale_chunk(b)
            fire_writeout(g, b)
        return None

    lax.fori_loop(0, _N_CHUNKS // 2, pair_body, None)
    wait_writeout(1)


@jax.jit
def _embed(x1d, table):
    mesh = plsc.VectorSubcoreMesh(core_axis_name="c", subcore_axis_name="s")
    fn = functools.partial(
        pl.kernel,
        mesh=mesh,
        out_type=jax.ShapeDtypeStruct((BATCH, SEQ, D_MODEL), jnp.float32),
        scratch_types=[
            pltpu.VMEM((_B_PER_W,), jnp.int32),
            pltpu.VMEM((2, _CHUNK, D_MODEL), jnp.float32),
            pltpu.SemaphoreType.DMA,
            pltpu.SemaphoreType.DMA,
            pltpu.SemaphoreType.DMA,
            pltpu.SemaphoreType.DMA,
        ],
        compiler_params=pltpu.CompilerParams(use_tc_tiling_on_sc=False),
    )(_emb_kernel)
    return fn(x1d, table)


def kernel(x, table):
    x1d = x.reshape(_B).astype(jnp.int32)
    return _embed(x1d, table)
